# baseline (speedup 1.0000x reference)
import jax
import jax.numpy as jnp
from jax import lax
from jax.experimental import pallas as pl
from jax.experimental.pallas import tpu as pltpu
from jax.experimental.pallas import tpu_sc as plsc

B = 4096
C = 26
NC = 1000
NWORK = 32
BPW = B // NWORK
KC = 200
KCH = NC // KC
NBLK = C * KCH
L = 16


def _sc_body(xt_hbm, out_hbm, idx_v, buf0, buf1, sem0, sem1):
    wid = lax.axis_index("s") * 2 + lax.axis_index("c")
    b0 = wid * BPW

    pltpu.sync_copy(xt_hbm.at[:, pl.ds(b0, BPW)], idx_v)

    zeros = jnp.zeros((L,), jnp.float32)
    ones = jnp.ones((L,), jnp.float32)
    lanes = lax.iota(jnp.int32, L)

    def _zero(buf):
        def body(r, _):
            for j in range(BPW // L):
                buf[r, pl.ds(j * L, L)] = zeros
            return 0
        lax.fori_loop(0, KC, body, 0)

    def _scatter(buf, n, val):
        c = n // KCH
        k0 = (n % KCH) * KC
        for j in range(BPW // L):
            v = idx_v[c, pl.ds(j * L, L)]
            kk = v - k0
            msk = (kk >= 0) & (kk < KC)
            plsc.store_scatter(buf, [kk, j * L + lanes], val, mask=msk)

    def _dma(buf, sem, n):
        c = n // KCH
        k0 = (n % KCH) * KC
        return pltpu.make_async_copy(
            buf, out_hbm.at[c, pl.ds(k0, KC), pl.ds(b0, BPW)], sem)

    _zero(buf0)
    _scatter(buf0, 0, ones)
    _dma(buf0, sem0, 0).start()
    _zero(buf1)
    _scatter(buf1, 1, ones)
    _dma(buf1, sem1, 1).start()

    def _pair(i, _):
        n0 = 2 * i
        _dma(buf0, sem0, n0 - 2).wait()
        _scatter(buf0, n0 - 2, zeros)
        _scatter(buf0, n0, ones)
        _dma(buf0, sem0, n0).start()
        n1 = 2 * i + 1
        _dma(buf1, sem1, n1 - 2).wait()
        _scatter(buf1, n1 - 2, zeros)
        _scatter(buf1, n1, ones)
        _dma(buf1, sem1, n1).start()
        return 0

    lax.fori_loop(1, NBLK // 2, _pair, 0)

    _dma(buf0, sem0, NBLK - 2).wait()
    _dma(buf1, sem1, NBLK - 1).wait()


@jax.jit
def kernel(x):
    mesh = plsc.VectorSubcoreMesh(core_axis_name="c", subcore_axis_name="s")
    run = pl.kernel(
        _sc_body,
        mesh=mesh,
        compiler_params=pltpu.CompilerParams(needs_layout_passes=False),
        out_type=jax.ShapeDtypeStruct((C, NC, B), jnp.float32),
        scratch_types=[
            pltpu.VMEM((C, BPW), jnp.int32),
            pltpu.VMEM((KC, BPW), jnp.float32),
            pltpu.VMEM((KC, BPW), jnp.float32),
            pltpu.SemaphoreType.DMA,
            pltpu.SemaphoreType.DMA,
        ],
    )
    out = run(x.T.astype(jnp.int32))
    return out.transpose(2, 0, 1)
